# baseline (device time: 24495 ns/iter reference)
import os

import jax
import jax.numpy as jnp
from jax import lax
from jax.experimental import pallas as pl
from jax.experimental.pallas import tpu as pltpu

N_DEV = 16
B, SQ, SKV, HQ_PER, DH = 2, 128, 128, 4, 64
D_MODEL = 512
ROWS = B * SQ
CH = ROWS // N_DEV
CH_PER_B = SQ // CH


def _project_q(x_ref, wq_ref):
    xb = x_ref[...].reshape(ROWS, D_MODEL).astype(jnp.bfloat16)
    wq = wq_ref[...].astype(jnp.bfloat16)
    q = lax.dot_general(xb, wq, (((1,), (0,)), ((), ())),
                        preferred_element_type=jnp.float32)
    return (q * 0.125).astype(jnp.bfloat16)


def _attn_batch(q, k_ref, v_ref, wo_ref, b):
    heads = []
    for h in range(HQ_PER):
        qb = q[b * SQ:(b + 1) * SQ, h * DH:(h + 1) * DH]
        kb = k_ref[b, h]
        vb = v_ref[b, h]
        s = lax.dot_general(qb, kb, (((1,), (1,)), ((), ())),
                            preferred_element_type=jnp.float32)
        qi = lax.broadcasted_iota(jnp.int32, (SQ, SKV), 0) // 64
        kj = lax.broadcasted_iota(jnp.int32, (SQ, SKV), 1) // 64
        s = jnp.where(kj <= qi, s, -1e9)
        m = jnp.max(s, axis=1, keepdims=True)
        w = jnp.exp(s - m)
        w = w / jnp.sum(w, axis=1, keepdims=True)
        heads.append(lax.dot_general(w.astype(jnp.bfloat16), vb,
                                     (((1,), (0,)), ((), ())),
                                     preferred_element_type=jnp.float32))
    ctx = jnp.concatenate(heads, axis=1).astype(jnp.bfloat16)
    return lax.dot_general(ctx, wo_ref[...].astype(jnp.bfloat16),
                           (((1,), (0,)), ((), ())),
                           preferred_element_type=jnp.float32
                           ).astype(jnp.bfloat16)


def kernel(x, Wq, K_ext, V_ext, Wo):
    my = lax.axis_index("i")
    h0 = my * HQ_PER
    K_sl = jnp.transpose(
        lax.dynamic_slice_in_dim(K_ext, h0, HQ_PER, 2).astype(jnp.bfloat16),
        (0, 2, 1, 3))
    V_sl = jnp.transpose(
        lax.dynamic_slice_in_dim(V_ext, h0, HQ_PER, 2).astype(jnp.bfloat16),
        (0, 2, 1, 3))

    _kmode = os.environ.get("KMODE", "full")
    _kbar = os.environ.get("KBAR", "explicit")

    def body(x_ref, wq_ref, k_ref, v_ref, wo_ref, out_ref,
             acc_ref, rs_buf, rs_send_sems, rs_recv_sems,
             ag_send_sems, ag_recv_sems):
        my_i = lax.axis_index("i")

        if _kmode == "compute":
            q = _project_q(x_ref, wq_ref)
            for b in range(B):
                acc_ref[pl.ds(b * SQ, SQ), :] = _attn_batch(q, k_ref, v_ref, wo_ref, b)
            out_ref[...] = acc_ref[...]
            return

        if _kbar == "explicit":
            barrier_sem = pltpu.get_barrier_semaphore()
            for j in range(N_DEV - 1):
                tgt = lax.rem(my_i + j + 1, N_DEV)
                pl.semaphore_signal(barrier_sem, inc=1, device_id=(tgt,),
                                    device_id_type=pl.DeviceIdType.MESH)
            pl.semaphore_wait(barrier_sem, N_DEV - 1)

        q = _project_q(x_ref, wq_ref)
        for b in range(B):
            acc_ref[pl.ds(b * SQ, SQ), :] = _attn_batch(q, k_ref, v_ref, wo_ref, b)
            for c in range(b * CH_PER_B, (b + 1) * CH_PER_B):
                @pl.when(c != my_i)
                def _send(c=c):
                    rdma = pltpu.make_async_remote_copy(
                        src_ref=acc_ref.at[pl.ds(c * CH, CH), :],
                        dst_ref=rs_buf.at[my_i],
                        send_sem=rs_send_sems.at[c],
                        recv_sem=rs_recv_sems.at[my_i],
                        device_id=(c,),
                        device_id_type=pl.DeviceIdType.MESH,
                    )
                    rdma.start()

                @pl.when(c == my_i)
                def _own(c=c):
                    rs_buf[my_i] = acc_ref[pl.ds(c * CH, CH), :]

        if _kmode == "barrier":
            out_ref[...] = acc_ref[...]
            return

        for s in range(N_DEV):
            @pl.when(s != my_i)
            def _wait(s=s):
                recv = pltpu.make_async_remote_copy(
                    src_ref=acc_ref.at[pl.ds(0, CH), :],
                    dst_ref=rs_buf.at[s],
                    send_sem=rs_send_sems.at[s],
                    recv_sem=rs_recv_sems.at[s],
                    device_id=(s,),
                    device_id_type=pl.DeviceIdType.MESH,
                )
                recv.wait_recv()

        red = jnp.sum(rs_buf[...].astype(jnp.float32), axis=0)
        out_ref[pl.ds(my_i * CH, CH), :] = red.astype(jnp.bfloat16)

        for tgt in range(N_DEV):
            @pl.when(tgt != my_i)
            def _send2(tgt=tgt):
                rdma = pltpu.make_async_remote_copy(
                    src_ref=out_ref.at[pl.ds(my_i * CH, CH), :],
                    dst_ref=out_ref.at[pl.ds(my_i * CH, CH), :],
                    send_sem=ag_send_sems.at[tgt],
                    recv_sem=ag_recv_sems.at[my_i],
                    device_id=(tgt,),
                    device_id_type=pl.DeviceIdType.MESH,
                )
                rdma.start()

        for s in range(N_DEV):
            @pl.when(s != my_i)
            def _wait2(s=s):
                recv = pltpu.make_async_remote_copy(
                    src_ref=out_ref.at[pl.ds(0, CH), :],
                    dst_ref=out_ref.at[pl.ds(s * CH, CH), :],
                    send_sem=ag_send_sems.at[s],
                    recv_sem=ag_recv_sems.at[s],
                    device_id=(s,),
                    device_id_type=pl.DeviceIdType.MESH,
                )
                recv.wait_recv()

        for c in range(N_DEV):
            @pl.when(c != my_i)
            def _drain(c=c):
                s1 = pltpu.make_async_remote_copy(
                    src_ref=acc_ref.at[pl.ds(c * CH, CH), :],
                    dst_ref=rs_buf.at[my_i],
                    send_sem=rs_send_sems.at[c],
                    recv_sem=rs_recv_sems.at[my_i],
                    device_id=(c,),
                    device_id_type=pl.DeviceIdType.MESH,
                )
                s1.wait_send()
                s2 = pltpu.make_async_remote_copy(
                    src_ref=out_ref.at[pl.ds(my_i * CH, CH), :],
                    dst_ref=out_ref.at[pl.ds(my_i * CH, CH), :],
                    send_sem=ag_send_sems.at[c],
                    recv_sem=ag_recv_sems.at[my_i],
                    device_id=(c,),
                    device_id_type=pl.DeviceIdType.MESH,
                )
                s2.wait_send()

    out = pl.pallas_call(
        body,
        out_shape=jax.ShapeDtypeStruct((ROWS, D_MODEL), jnp.bfloat16),
        in_specs=[pl.BlockSpec(memory_space=pltpu.VMEM)] * 5,
        out_specs=pl.BlockSpec(memory_space=pltpu.VMEM),
        scratch_shapes=[
            pltpu.VMEM((ROWS, D_MODEL), jnp.bfloat16),
            pltpu.VMEM((N_DEV, CH, D_MODEL), jnp.bfloat16),
            pltpu.SemaphoreType.DMA((N_DEV,)),
            pltpu.SemaphoreType.DMA((N_DEV,)),
            pltpu.SemaphoreType.DMA((N_DEV,)),
            pltpu.SemaphoreType.DMA((N_DEV,)),
        ],
        compiler_params=pltpu.CompilerParams(collective_id=0),
    )(x, Wq, K_sl, V_sl, Wo)
    return out.reshape(B, SQ, D_MODEL)


# device time: 24072 ns/iter; 1.0176x vs baseline; 1.0176x over previous
import os

import jax
import jax.numpy as jnp
from jax import lax
from jax.experimental import pallas as pl
from jax.experimental.pallas import tpu as pltpu

N_DEV = 16
B, SQ, SKV, HQ_PER, DH = 2, 128, 128, 4, 64
D_MODEL = 512
ROWS = B * SQ
CH = ROWS // N_DEV
CH_PER_B = SQ // CH


def _project_q(x_ref, wq_ref):
    xb = x_ref[...].reshape(ROWS, D_MODEL).astype(jnp.bfloat16)
    wq = wq_ref[...].astype(jnp.bfloat16)
    q = lax.dot_general(xb, wq, (((1,), (0,)), ((), ())),
                        preferred_element_type=jnp.float32)
    return (q * 0.125).astype(jnp.bfloat16)


def _attn_batch(q, k_ref, v_ref, wo_ref, b):
    kf = k_ref[b]
    vf = v_ref[b]
    heads = []
    for h in range(HQ_PER):
        qb = q[b * SQ:(b + 1) * SQ, h * DH:(h + 1) * DH]
        kb = kf[:, h * DH:(h + 1) * DH]
        vb = vf[:, h * DH:(h + 1) * DH]
        s = lax.dot_general(qb, kb, (((1,), (1,)), ((), ())),
                            preferred_element_type=jnp.float32)
        qi = lax.broadcasted_iota(jnp.int32, (SQ, SKV), 0) // 64
        kj = lax.broadcasted_iota(jnp.int32, (SQ, SKV), 1) // 64
        w = jnp.where(kj <= qi, jnp.exp(s), 0.0)
        w = w * (1.0 / jnp.sum(w, axis=1, keepdims=True))
        heads.append(lax.dot_general(w.astype(jnp.bfloat16), vb,
                                     (((1,), (0,)), ((), ())),
                                     preferred_element_type=jnp.float32))
    ctx = jnp.concatenate(heads, axis=1).astype(jnp.bfloat16)
    return lax.dot_general(ctx, wo_ref[...].astype(jnp.bfloat16),
                           (((1,), (0,)), ((), ())),
                           preferred_element_type=jnp.float32
                           ).astype(jnp.bfloat16)


def kernel(x, Wq, K_ext, V_ext, Wo):
    my = lax.axis_index("i")
    h0 = my * HQ_PER
    K_sl = lax.dynamic_slice_in_dim(K_ext, h0, HQ_PER, 2).astype(
        jnp.bfloat16).reshape(B, SKV, HQ_PER * DH)
    V_sl = lax.dynamic_slice_in_dim(V_ext, h0, HQ_PER, 2).astype(
        jnp.bfloat16).reshape(B, SKV, HQ_PER * DH)

    _kmode = os.environ.get("KMODE", "full")
    _kbar = os.environ.get("KBAR", "explicit")

    def body(x_ref, wq_ref, k_ref, v_ref, wo_ref, out_ref,
             acc_ref, rs_buf, rs_send_sems, rs_recv_sems,
             ag_send_sems, ag_recv_sems):
        my_i = lax.axis_index("i")

        if _kmode == "compute":
            q = _project_q(x_ref, wq_ref)
            for b in range(B):
                acc_ref[pl.ds(b * SQ, SQ), :] = _attn_batch(q, k_ref, v_ref, wo_ref, b)
            out_ref[...] = acc_ref[...]
            return

        if _kbar == "explicit":
            barrier_sem = pltpu.get_barrier_semaphore()
            for j in range(N_DEV - 1):
                tgt = lax.rem(my_i + j + 1, N_DEV)
                pl.semaphore_signal(barrier_sem, inc=1, device_id=(tgt,),
                                    device_id_type=pl.DeviceIdType.MESH)
            pl.semaphore_wait(barrier_sem, N_DEV - 1)

        q = _project_q(x_ref, wq_ref)
        for b in range(B):
            acc_ref[pl.ds(b * SQ, SQ), :] = _attn_batch(q, k_ref, v_ref, wo_ref, b)
            for c in range(b * CH_PER_B, (b + 1) * CH_PER_B):
                @pl.when(c != my_i)
                def _send(c=c):
                    rdma = pltpu.make_async_remote_copy(
                        src_ref=acc_ref.at[pl.ds(c * CH, CH), :],
                        dst_ref=rs_buf.at[my_i],
                        send_sem=rs_send_sems.at[c],
                        recv_sem=rs_recv_sems.at[my_i],
                        device_id=(c,),
                        device_id_type=pl.DeviceIdType.MESH,
                    )
                    rdma.start()

                @pl.when(c == my_i)
                def _own(c=c):
                    rs_buf[my_i] = acc_ref[pl.ds(c * CH, CH), :]

        if _kmode == "barrier":
            out_ref[...] = acc_ref[...]
            return

        for s in range(N_DEV):
            @pl.when(s != my_i)
            def _wait(s=s):
                recv = pltpu.make_async_remote_copy(
                    src_ref=acc_ref.at[pl.ds(0, CH), :],
                    dst_ref=rs_buf.at[s],
                    send_sem=rs_send_sems.at[s],
                    recv_sem=rs_recv_sems.at[s],
                    device_id=(s,),
                    device_id_type=pl.DeviceIdType.MESH,
                )
                recv.wait_recv()

        red = jnp.sum(rs_buf[...].astype(jnp.float32), axis=0)
        out_ref[pl.ds(my_i * CH, CH), :] = red.astype(jnp.bfloat16)

        for tgt in range(N_DEV):
            @pl.when(tgt != my_i)
            def _send2(tgt=tgt):
                rdma = pltpu.make_async_remote_copy(
                    src_ref=out_ref.at[pl.ds(my_i * CH, CH), :],
                    dst_ref=out_ref.at[pl.ds(my_i * CH, CH), :],
                    send_sem=ag_send_sems.at[tgt],
                    recv_sem=ag_recv_sems.at[my_i],
                    device_id=(tgt,),
                    device_id_type=pl.DeviceIdType.MESH,
                )
                rdma.start()

        for s in range(N_DEV):
            @pl.when(s != my_i)
            def _wait2(s=s):
                recv = pltpu.make_async_remote_copy(
                    src_ref=out_ref.at[pl.ds(0, CH), :],
                    dst_ref=out_ref.at[pl.ds(s * CH, CH), :],
                    send_sem=ag_send_sems.at[s],
                    recv_sem=ag_recv_sems.at[s],
                    device_id=(s,),
                    device_id_type=pl.DeviceIdType.MESH,
                )
                recv.wait_recv()

        for c in range(N_DEV):
            @pl.when(c != my_i)
            def _drain(c=c):
                s1 = pltpu.make_async_remote_copy(
                    src_ref=acc_ref.at[pl.ds(c * CH, CH), :],
                    dst_ref=rs_buf.at[my_i],
                    send_sem=rs_send_sems.at[c],
                    recv_sem=rs_recv_sems.at[my_i],
                    device_id=(c,),
                    device_id_type=pl.DeviceIdType.MESH,
                )
                s1.wait_send()
                s2 = pltpu.make_async_remote_copy(
                    src_ref=out_ref.at[pl.ds(my_i * CH, CH), :],
                    dst_ref=out_ref.at[pl.ds(my_i * CH, CH), :],
                    send_sem=ag_send_sems.at[c],
                    recv_sem=ag_recv_sems.at[my_i],
                    device_id=(c,),
                    device_id_type=pl.DeviceIdType.MESH,
                )
                s2.wait_send()

    out = pl.pallas_call(
        body,
        out_shape=jax.ShapeDtypeStruct((ROWS, D_MODEL), jnp.bfloat16),
        in_specs=[pl.BlockSpec(memory_space=pltpu.VMEM)] * 5,
        out_specs=pl.BlockSpec(memory_space=pltpu.VMEM),
        scratch_shapes=[
            pltpu.VMEM((ROWS, D_MODEL), jnp.bfloat16),
            pltpu.VMEM((N_DEV, CH, D_MODEL), jnp.bfloat16),
            pltpu.SemaphoreType.DMA((N_DEV,)),
            pltpu.SemaphoreType.DMA((N_DEV,)),
            pltpu.SemaphoreType.DMA((N_DEV,)),
            pltpu.SemaphoreType.DMA((N_DEV,)),
        ],
        compiler_params=pltpu.CompilerParams(collective_id=0),
    )(x, Wq, K_sl, V_sl, Wo)
    return out.reshape(B, SQ, D_MODEL)
